# baseline (speedup 1.0000x reference)
import functools

import jax
import jax.numpy as jnp
from jax import lax
from jax.experimental import pallas as pl
from jax.experimental.pallas import tpu as pltpu
from jax.experimental.pallas import tpu_sc as plsc

_WS = 512
_GR = 8
_LANES = 16


def _len_body(mask_ref, v_ref, maxlen_ref, mo_ref, vo_ref):
    m = mask_ref[...]
    maxlen = jnp.maximum(jnp.max(jnp.sum(m, axis=-1)), 1)
    maxlen_ref[0] = maxlen
    L = m.shape[-1]
    keep = jax.lax.broadcasted_iota(jnp.int32, (1, L), 1) < maxlen
    mo_ref[...] = jnp.logical_and(keep, m != 0)
    vo_ref[...] = jnp.where(keep, v_ref[...], 0.0)


def _make_sc_trim(R, L):
    n_strips = L // _WS
    mesh = plsc.VectorSubcoreMesh(core_axis_name="c", subcore_axis_name="s")
    num_cores, num_subcores = 2, 16
    nw = num_cores * num_subcores
    rows_per_w = R // nw
    n_groups = rows_per_w // _GR

    @functools.partial(
        pl.kernel,
        out_type=jax.ShapeDtypeStruct((R, L), jnp.float32),
        mesh=mesh,
        scratch_types=[
            pltpu.VMEM((_GR, L), jnp.float32),
            pltpu.VMEM((_GR, L), jnp.float32),
            pltpu.VMEM((_GR, L), jnp.float32),
            pltpu.VMEM((16,), jnp.int32),
            pltpu.SemaphoreType.DMA,
            pltpu.SemaphoreType.DMA,
        ],
    )
    def sc_trim(x_hbm, mlen_hbm, out_hbm, buf_a, buf_b, zbuf, mlen_v,
                rd_sem, wr_sem):
        wid = lax.axis_index("s") * num_cores + lax.axis_index("c")
        row_base = wid * rows_per_w

        pltpu.sync_copy(mlen_hbm, mlen_v)
        mlen = mlen_v[...][0]
        nfull = mlen // _WS
        nread = jnp.minimum(nfull + 1, n_strips)
        cs = jnp.minimum((mlen // _LANES) * _LANES, L - _LANES)
        col = cs + lax.iota(jnp.int32, _LANES)
        keepv = col < mlen
        zero = jnp.zeros((_LANES,), jnp.float32)

        for r in range(_GR):
            def _z(ci, _, r=r):
                zbuf[r, pl.ds(ci * _LANES, _LANES)] = zero
                return _
            lax.fori_loop(0, L // _LANES, _z, 0)

        bufs = [buf_a, buf_b]

        def fire_reads(g, buf):
            row0 = row_base + g * _GR

            def _rd(s, _):
                pltpu.async_copy(
                    x_hbm.at[pl.ds(row0, _GR), pl.ds(s * _WS, _WS)],
                    buf.at[:, pl.ds(s * _WS, _WS)], rd_sem)
                return _
            lax.fori_loop(0, nread, _rd, 0)

        def drain(sem, count, buf):
            def _w(s, _):
                pltpu.make_async_copy(
                    x_hbm.at[pl.ds(0, _GR), pl.ds(0, _WS)],
                    buf.at[:, pl.ds(0, _WS)], sem).wait()
                return _
            lax.fori_loop(0, count, _w, 0)

        def fire_writes(g, buf):
            row0 = row_base + g * _GR

            def _wl(s, _):
                pltpu.async_copy(
                    buf.at[:, pl.ds(s * _WS, _WS)],
                    out_hbm.at[pl.ds(row0, _GR), pl.ds(s * _WS, _WS)],
                    wr_sem)
                return _
            lax.fori_loop(0, nread, _wl, 0)

            def _wz(s, _):
                pltpu.async_copy(
                    zbuf.at[:, pl.ds(s * _WS, _WS)],
                    out_hbm.at[pl.ds(row0, _GR), pl.ds(s * _WS, _WS)],
                    wr_sem)
                return _
            lax.fori_loop(nread, n_strips, _wz, 0)

        fire_reads(0, bufs[0])
        for g in range(n_groups):
            cur = bufs[g % 2]
            nxt = bufs[(g + 1) % 2]
            if g >= 1:
                drain(wr_sem, n_strips, cur)
            if g + 1 < n_groups:
                fire_reads(g + 1, nxt)
            drain(rd_sem, nread, cur)
            for r in range(_GR):
                vec = cur[r, pl.ds(cs, _LANES)]
                cur[r, pl.ds(cs, _LANES)] = jnp.where(keepv, vec, 0.0)

                def _zt(ci, _, r=r):
                    cur[r, pl.ds(ci * _LANES, _LANES)] = zero
                    return _
                lax.fori_loop(cs // _LANES + 1, nread * (_WS // _LANES),
                              _zt, 0)
            fire_writes(g, cur)
        drain(wr_sem, n_strips, bufs[(n_groups - 1) % 2])

    return sc_trim


def kernel(x, v, mask):
    B, C, L = x.shape
    Cv = v.shape[1]
    x2 = x.reshape(B * C, L)
    v2 = v.reshape(B * Cv, L)
    m2 = mask.reshape(B, L)

    maxlen, m_out2, v_out2 = pl.pallas_call(
        _len_body,
        in_specs=[
            pl.BlockSpec((B, L), lambda: (0, 0)),
            pl.BlockSpec((B * Cv, L), lambda: (0, 0)),
        ],
        out_specs=[
            pl.BlockSpec(memory_space=pltpu.SMEM),
            pl.BlockSpec((B, L), lambda: (0, 0)),
            pl.BlockSpec((B * Cv, L), lambda: (0, 0)),
        ],
        out_shape=[
            jax.ShapeDtypeStruct((1,), jnp.int32),
            jax.ShapeDtypeStruct((B, L), jnp.bool_),
            jax.ShapeDtypeStruct((B * Cv, L), v.dtype),
        ],
    )(m2, v2)

    mlen16 = jnp.broadcast_to(maxlen, (16,))
    x_out2 = _make_sc_trim(B * C, L)(x2, mlen16)

    return (
        x_out2.reshape(B, C, L),
        v_out2.reshape(B, Cv, L),
        m_out2.reshape(B, 1, L),
    )


# speedup vs baseline: 1.0421x; 1.0421x over previous
import functools

import jax
import jax.numpy as jnp
from jax import lax
from jax.experimental import pallas as pl
from jax.experimental.pallas import tpu as pltpu
from jax.experimental.pallas import tpu_sc as plsc

_ROWS = 512
_LANES = 16


def _x_body(x_ref, mask_ref, xo_ref, mo_ref, maxlen_ref):
    i = pl.program_id(0)
    L = x_ref.shape[-1]

    @pl.when(i == 0)
    def _prologue():
        m = mask_ref[...]
        maxlen = jnp.maximum(jnp.max(jnp.sum(m, axis=-1)), 1)
        maxlen_ref[0] = maxlen
        keep_row = jax.lax.broadcasted_iota(jnp.int32, (1, L), 1) < maxlen
        mo_ref[...] = jnp.logical_and(keep_row, m != 0)

    maxlen = maxlen_ref[0]
    keep = jax.lax.broadcasted_iota(jnp.int32, x_ref.shape, 1) < maxlen
    xo_ref[...] = jnp.where(keep, x_ref[...], 0.0)


def _make_sc_vtrim(B, Rv, L):
    mesh = plsc.VectorSubcoreMesh(core_axis_name="c", subcore_axis_name="s")
    num_cores = 2
    nw = num_cores * 16
    rows_per_w = Rv // nw
    n_slices = L // _LANES

    @functools.partial(
        pl.kernel,
        out_type=jax.ShapeDtypeStruct((Rv, L), jnp.float32),
        mesh=mesh,
        scratch_types=[
            pltpu.VMEM((B, L), jnp.int32),
            pltpu.VMEM((rows_per_w, L), jnp.float32),
            pltpu.VMEM((2 * _LANES,), jnp.int32),
        ],
    )
    def sc_vtrim(v_hbm, mask_hbm, out_hbm, mbuf, vbuf, accbuf):
        wid = lax.axis_index("s") * num_cores + lax.axis_index("c")
        row0 = wid * rows_per_w

        pltpu.sync_copy(mask_hbm, mbuf)
        pltpu.sync_copy(v_hbm.at[pl.ds(row0, rows_per_w)], vbuf)

        iota = lax.iota(jnp.int32, _LANES)
        zvec = jnp.zeros((_LANES,), jnp.int32)

        def row_sum(r, rmax):
            accbuf[pl.ds(0, _LANES)] = zvec

            def acc_fn(ci, _):
                accbuf[pl.ds(0, _LANES)] = (
                    accbuf[pl.ds(0, _LANES)]
                    + mbuf[r, pl.ds(ci * _LANES, _LANES)])
                return _
            lax.fori_loop(0, n_slices, acc_fn, 0)
            acc = accbuf[pl.ds(0, _LANES)]
            total = acc[0]
            for l in range(1, _LANES):
                total = total + acc[l]
            return jnp.maximum(rmax, total)

        maxlen = jnp.maximum(lax.fori_loop(0, B, row_sum, 0), 1)

        for r in range(rows_per_w):
            def trim_fn(ci, _, r=r):
                col = ci * _LANES + iota
                vec = vbuf[r, pl.ds(ci * _LANES, _LANES)]
                vbuf[r, pl.ds(ci * _LANES, _LANES)] = jnp.where(
                    col < maxlen, vec, 0.0)
                return _
            lax.fori_loop(0, n_slices, trim_fn, 0)

        pltpu.sync_copy(vbuf, out_hbm.at[pl.ds(row0, rows_per_w)])

    return sc_vtrim


def kernel(x, v, mask):
    B, C, L = x.shape
    Cv = v.shape[1]
    x2 = x.reshape(B * C, L)
    v2 = v.reshape(B * Cv, L)
    m2 = mask.reshape(B, L)
    n_blocks = (B * C) // _ROWS

    x_out2, m_out2 = pl.pallas_call(
        _x_body,
        grid=(n_blocks,),
        in_specs=[
            pl.BlockSpec((_ROWS, L), lambda i: (i, 0)),
            pl.BlockSpec((B, L), lambda i: (0, 0)),
        ],
        out_specs=[
            pl.BlockSpec((_ROWS, L), lambda i: (i, 0)),
            pl.BlockSpec((B, L), lambda i: (0, 0)),
        ],
        out_shape=[
            jax.ShapeDtypeStruct((B * C, L), x.dtype),
            jax.ShapeDtypeStruct((B, L), jnp.bool_),
        ],
        scratch_shapes=[pltpu.SMEM((1,), jnp.int32)],
    )(x2, m2)

    v_out2 = _make_sc_vtrim(B, B * Cv, L)(v2, m2)

    return (
        x_out2.reshape(B, C, L),
        v_out2.reshape(B, Cv, L),
        m_out2.reshape(B, 1, L),
    )


# speedup vs baseline: 1.4582x; 1.3993x over previous
import jax
import jax.numpy as jnp
from jax.experimental import pallas as pl
from jax.experimental.pallas import tpu as pltpu

_ROWS = 256


def _trim_body(x_ref, v_ref, mask_ref, xo_ref, vo_ref, mo_ref, maxlen_ref):
    i = pl.program_id(0)
    L = x_ref.shape[-1]

    @pl.when(i == 0)
    def _prologue():
        m = mask_ref[...]
        maxlen = jnp.maximum(jnp.max(jnp.sum(m, axis=-1)), 1)
        maxlen_ref[0] = maxlen
        keep_row = jax.lax.broadcasted_iota(jnp.int32, (1, L), 1) < maxlen
        mo_ref[...] = jnp.logical_and(keep_row, m != 0)
        vo_ref[...] = jnp.where(keep_row, v_ref[...], 0.0)

    maxlen = maxlen_ref[0]
    keep = jax.lax.broadcasted_iota(jnp.int32, x_ref.shape, 1) < maxlen
    xo_ref[...] = jnp.where(keep, x_ref[...], 0.0)


def kernel(x, v, mask):
    B, C, L = x.shape
    Cv = v.shape[1]
    x2 = x.reshape(B * C, L)
    v2 = v.reshape(B * Cv, L)
    m2 = mask.reshape(B, L)
    n_blocks = (B * C) // _ROWS

    x_out2, v_out2, m_out2 = pl.pallas_call(
        _trim_body,
        grid=(n_blocks,),
        in_specs=[
            pl.BlockSpec((_ROWS, L), lambda i: (i, 0)),
            pl.BlockSpec((B * Cv, L), lambda i: (0, 0)),
            pl.BlockSpec((B, L), lambda i: (0, 0)),
        ],
        out_specs=[
            pl.BlockSpec((_ROWS, L), lambda i: (i, 0)),
            pl.BlockSpec((B * Cv, L), lambda i: (0, 0)),
            pl.BlockSpec((B, L), lambda i: (0, 0)),
        ],
        out_shape=[
            jax.ShapeDtypeStruct((B * C, L), x.dtype),
            jax.ShapeDtypeStruct((B * Cv, L), v.dtype),
            jax.ShapeDtypeStruct((B, L), jnp.bool_),
        ],
        scratch_shapes=[pltpu.SMEM((1,), jnp.int32)],
    )(x2, v2, m2)

    return (
        x_out2.reshape(B, C, L),
        v_out2.reshape(B, Cv, L),
        m_out2.reshape(B, 1, L),
    )


# speedup vs baseline: 1.5052x; 1.0322x over previous
import jax
import jax.numpy as jnp
from jax.experimental import pallas as pl
from jax.experimental.pallas import tpu as pltpu

_ROWS = 512


def _trim_body(x_ref, v_ref, mask_ref, xo_ref, vo_ref, mo_ref, maxlen_ref):
    i = pl.program_id(0)
    L = x_ref.shape[-1]

    @pl.when(i == 0)
    def _prologue():
        m = mask_ref[...]
        maxlen = jnp.maximum(jnp.max(jnp.sum(m, axis=-1)), 1)
        maxlen_ref[0] = maxlen
        keep_row = jax.lax.broadcasted_iota(jnp.int32, (1, L), 1) < maxlen
        mo_ref[...] = jnp.logical_and(keep_row, m != 0)
        vo_ref[...] = jnp.where(keep_row, v_ref[...], 0.0)

    maxlen = maxlen_ref[0]
    keep = jax.lax.broadcasted_iota(jnp.int32, x_ref.shape, 1) < maxlen
    xo_ref[...] = jnp.where(keep, x_ref[...], 0.0)


def kernel(x, v, mask):
    B, C, L = x.shape
    Cv = v.shape[1]
    x2 = x.reshape(B * C, L)
    v2 = v.reshape(B * Cv, L)
    m2 = mask.reshape(B, L)
    n_blocks = (B * C) // _ROWS

    x_out2, v_out2, m_out2 = pl.pallas_call(
        _trim_body,
        grid=(n_blocks,),
        in_specs=[
            pl.BlockSpec((_ROWS, L), lambda i: (i, 0)),
            pl.BlockSpec((B * Cv, L), lambda i: (0, 0)),
            pl.BlockSpec((B, L), lambda i: (0, 0)),
        ],
        out_specs=[
            pl.BlockSpec((_ROWS, L), lambda i: (i, 0)),
            pl.BlockSpec((B * Cv, L), lambda i: (0, 0)),
            pl.BlockSpec((B, L), lambda i: (0, 0)),
        ],
        out_shape=[
            jax.ShapeDtypeStruct((B * C, L), x.dtype),
            jax.ShapeDtypeStruct((B * Cv, L), v.dtype),
            jax.ShapeDtypeStruct((B, L), jnp.bool_),
        ],
        scratch_shapes=[pltpu.SMEM((1,), jnp.int32)],
    )(x2, v2, m2)

    return (
        x_out2.reshape(B, C, L),
        v_out2.reshape(B, Cv, L),
        m_out2.reshape(B, 1, L),
    )
